# Initial kernel scaffold; baseline (speedup 1.0000x reference)
import jax
import jax.numpy as jnp
from jax.experimental import pallas as pl


def kernel(logits, temperatures, exponential):
    raise NotImplementedError("write your pallas kernel here")


# speedup vs baseline: 3.8975x; 3.8975x over previous
import functools

import jax
import jax.numpy as jnp
from jax.experimental import pallas as pl
from jax.experimental.pallas import tpu as pltpu

TOKENS = 32
VOCAB = 1000000
EPS = 1e-10
CHUNK = 65536
GRID = (VOCAB + CHUNK - 1) // CHUNK


def _sample_kernel(x_ref, e_ref, scale_ref, nw_ref, o_ref, m_ref):
    i = pl.program_id(0)

    @pl.when(i == 0)
    def _init():
        m_ref[...] = jnp.full((TOKENS, 1), -jnp.inf, jnp.float32)
        o_ref[...] = jnp.zeros((TOKENS, 1), jnp.int32)

    x = x_ref[...]
    e = e_ref[...]
    scale = scale_ref[...]
    nw = nw_ref[...]

    noise = jnp.log(e + EPS)
    key = x * scale - nw * noise

    idx = jax.lax.broadcasted_iota(jnp.int32, key.shape, 1)
    valid = (idx + i * CHUNK) < VOCAB
    key = jnp.where(valid, key, -jnp.inf)

    loc_max = jnp.max(key, axis=1, keepdims=True)
    loc_arg = jnp.argmax(key, axis=1).astype(jnp.int32)[:, None] + i * CHUNK

    better = loc_max > m_ref[...]
    m_ref[...] = jnp.where(better, loc_max, m_ref[...])
    o_ref[...] = jnp.where(better, loc_arg, o_ref[...])


@functools.partial(jax.jit, static_argnames=())
def kernel(logits, temperatures, exponential):
    t = temperatures[:, None].astype(jnp.float32)
    pos = t > 0
    scale = jnp.where(pos, 1.0 / jnp.where(pos, t, 1.0), 1.0)
    nw = jnp.where(pos, 1.0, 0.0)

    out = pl.pallas_call(
        _sample_kernel,
        grid=(GRID,),
        in_specs=[
            pl.BlockSpec((TOKENS, CHUNK), lambda i: (0, i)),
            pl.BlockSpec((1, CHUNK), lambda i: (0, i)),
            pl.BlockSpec((TOKENS, 1), lambda i: (0, 0)),
            pl.BlockSpec((TOKENS, 1), lambda i: (0, 0)),
        ],
        out_specs=pl.BlockSpec((TOKENS, 1), lambda i: (0, 0)),
        out_shape=jax.ShapeDtypeStruct((TOKENS, 1), jnp.int32),
        scratch_shapes=[pltpu.VMEM((TOKENS, 1), jnp.float32)],
    )(logits, exponential, scale, nw)
    return out[:, 0]


# speedup vs baseline: 5.0896x; 1.3059x over previous
import jax
import jax.numpy as jnp
from jax.experimental import pallas as pl
from jax.experimental.pallas import tpu as pltpu

TOKENS = 32
VOCAB = 1000000
EPS = 1e-10
CHUNK = 65536
GRID = (VOCAB + CHUNK - 1) // CHUNK


def _sample_kernel(x_ref, e_ref, t_ref, o_ref, m_ref):
    i = pl.program_id(0)

    @pl.when(i == 0)
    def _init():
        m_ref[...] = jnp.full((TOKENS, 1), -jnp.inf, jnp.float32)
        o_ref[...] = jnp.zeros((TOKENS, 1), jnp.int32)

    x = x_ref[...]
    e = e_ref[...]
    t = t_ref[...]

    noise = jnp.log(e + EPS)
    key = x - t * noise

    idx = jax.lax.broadcasted_iota(jnp.int32, key.shape, 1)
    key = jnp.where(idx < VOCAB - i * CHUNK, key, -jnp.inf)

    loc_max = jnp.max(key, axis=1, keepdims=True)
    hit = key == loc_max
    loc_arg = jnp.min(jnp.where(hit, idx, VOCAB), axis=1, keepdims=True)
    loc_arg = loc_arg + i * CHUNK

    better = loc_max > m_ref[...]
    m_ref[...] = jnp.where(better, loc_max, m_ref[...])
    o_ref[...] = jnp.where(better, loc_arg, o_ref[...])


@jax.jit
def kernel(logits, temperatures, exponential):
    t = temperatures[:, None].astype(jnp.float32)
    out = pl.pallas_call(
        _sample_kernel,
        grid=(GRID,),
        in_specs=[
            pl.BlockSpec((TOKENS, CHUNK), lambda i: (0, i)),
            pl.BlockSpec((1, CHUNK), lambda i: (0, i)),
            pl.BlockSpec((TOKENS, 1), lambda i: (0, 0)),
        ],
        out_specs=pl.BlockSpec((TOKENS, 1), lambda i: (0, 0)),
        out_shape=jax.ShapeDtypeStruct((TOKENS, 1), jnp.int32),
        scratch_shapes=[pltpu.VMEM((TOKENS, 1), jnp.float32)],
    )(logits, exponential, t)
    return out[:, 0]
